# Initial kernel scaffold; baseline (speedup 1.0000x reference)
import jax
import jax.numpy as jnp
from jax.experimental import pallas as pl


def kernel(features, weight, bias):
    raise NotImplementedError("write your pallas kernel here")


# speedup vs baseline: 3.1866x; 3.1866x over previous
import jax
import jax.numpy as jnp
from jax.experimental import pallas as pl
from jax.experimental.pallas import tpu as pltpu

_NB = 2


def _head_kernel(x_ref, w_ref, b_ref, o_ref):
    w = w_ref[...]
    b = b_ref[...]
    for i in range(x_ref.shape[0]):
        x = x_ref[i].astype(jnp.bfloat16)
        y = jnp.dot(w, x, preferred_element_type=jnp.float32)
        o_ref[i] = (y + b).astype(o_ref.dtype)


def kernel(features, weight, bias):
    B, T, Cin, fh, fw = features.shape
    Cout = weight.shape[0]
    N = B * T
    HW = fh * fw

    x = features.reshape(N, Cin, HW)
    wb = weight.astype(jnp.bfloat16)
    b2d = bias.reshape(Cout, 1)

    nb = _NB if N % _NB == 0 else 1
    out = pl.pallas_call(
        _head_kernel,
        out_shape=jax.ShapeDtypeStruct((N, Cout, HW), features.dtype),
        grid=(N // nb,),
        in_specs=[
            pl.BlockSpec((nb, Cin, HW), lambda n: (n, 0, 0)),
            pl.BlockSpec((Cout, Cin), lambda n: (0, 0)),
            pl.BlockSpec((Cout, 1), lambda n: (0, 0)),
        ],
        out_specs=pl.BlockSpec((nb, Cout, HW), lambda n: (n, 0, 0)),
        compiler_params=pltpu.CompilerParams(
            dimension_semantics=("parallel",)),
        name="conv1x1_head",
    )(x, wb, b2d)
    return out.reshape(B, T, Cout, fh, fw)


# speedup vs baseline: 26.7055x; 8.3805x over previous
import jax
import jax.numpy as jnp
from jax.experimental import pallas as pl
from jax.experimental.pallas import tpu as pltpu

_CN = 256


def _head_kernel(x_ref, w_ref, b_ref, o_ref):
    fh, fw, t, cin = x_ref.shape[1:]
    m = fh * fw * t
    cout = w_ref.shape[0]
    x2 = x_ref[0].astype(jnp.bfloat16).reshape(m, cin)
    for c in range(0, cout, _CN):
        wc = w_ref[c:c + _CN, :]
        y = jax.lax.dot_general(
            x2, wc, (((1,), (1,)), ((), ())),
            preferred_element_type=jnp.float32)
        y = y + b_ref[:, c:c + _CN]
        o_ref[0, :, :, :, c:c + _CN] = y.reshape(fh, fw, t, _CN)


def kernel(features, weight, bias):
    B, T, Cin, fh, fw = features.shape
    Cout = weight.shape[0]

    xp = jnp.transpose(features, (0, 3, 4, 1, 2))
    wb = weight.astype(jnp.bfloat16)
    b2d = bias.reshape(1, Cout)

    out = pl.pallas_call(
        _head_kernel,
        out_shape=jax.ShapeDtypeStruct((B, fh, fw, T, Cout), features.dtype),
        grid=(B,),
        in_specs=[
            pl.BlockSpec((1, fh, fw, T, Cin), lambda b: (b, 0, 0, 0, 0)),
            pl.BlockSpec((Cout, Cin), lambda b: (0, 0)),
            pl.BlockSpec((1, Cout), lambda b: (0, 0)),
        ],
        out_specs=pl.BlockSpec((1, fh, fw, T, Cout), lambda b: (b, 0, 0, 0, 0)),
        compiler_params=pltpu.CompilerParams(
            dimension_semantics=("parallel",)),
        name="conv1x1_head",
    )(xp, wb, b2d)
    return jnp.transpose(out, (0, 3, 4, 1, 2))


# speedup vs baseline: 30.1838x; 1.1302x over previous
import jax
import jax.numpy as jnp
from jax.experimental import pallas as pl
from jax.experimental.pallas import tpu as pltpu

_CN = 256


def _head_kernel(x_ref, w_ref, b_ref, o_ref):
    fh, fw, t, cin = x_ref.shape[1:]
    m = fh * fw * t
    cout = w_ref.shape[0]
    x2 = x_ref[0].astype(jnp.bfloat16).reshape(m, cin)
    for c in range(0, cout, _CN):
        wc = w_ref[c:c + _CN, :].astype(jnp.bfloat16)
        y = jax.lax.dot_general(
            x2, wc, (((1,), (1,)), ((), ())),
            preferred_element_type=jnp.float32)
        y = y + b_ref[:, c:c + _CN]
        o_ref[0, :, :, :, c:c + _CN] = y.reshape(fh, fw, t, _CN)


def kernel(features, weight, bias):
    B, T, Cin, fh, fw = features.shape
    Cout = weight.shape[0]

    xp = jnp.transpose(features, (0, 3, 4, 1, 2))
    b2d = bias.reshape(1, Cout)

    out = pl.pallas_call(
        _head_kernel,
        out_shape=jax.ShapeDtypeStruct((B, fh, fw, T, Cout), features.dtype),
        grid=(B,),
        in_specs=[
            pl.BlockSpec((1, fh, fw, T, Cin), lambda b: (b, 0, 0, 0, 0)),
            pl.BlockSpec((Cout, Cin), lambda b: (0, 0)),
            pl.BlockSpec((1, Cout), lambda b: (0, 0)),
        ],
        out_specs=pl.BlockSpec((1, fh, fw, T, Cout), lambda b: (b, 0, 0, 0, 0)),
        compiler_params=pltpu.CompilerParams(
            dimension_semantics=("parallel",)),
        name="conv1x1_head",
    )(xp, weight, b2d)
    return jnp.transpose(out, (0, 3, 4, 1, 2))
